# Initial kernel scaffold; baseline (speedup 1.0000x reference)
import jax
import jax.numpy as jnp
from jax.experimental import pallas as pl


def kernel(x_flat, idx_flat, scores_flat, capacity, W1, B1, W2, B2):
    raise NotImplementedError("write your pallas kernel here")


# speedup vs baseline: 2.0105x; 2.0105x over previous
import functools

import jax
import jax.numpy as jnp
from jax import lax
from jax.experimental import pallas as pl
from jax.experimental.pallas import tpu as pltpu, tpu_sc as plsc

E = 8
T = 4096
K = 2
D = 1024
F = 2048
CAP = 1024
A = T * K

NC, NS, L = 2, 16, 16
NW = NC * NS

_MESH = dict(core_axis_name="c", subcore_axis_name="s",
             num_cores=NC, num_subcores=NS)


_AW = A // NS
_VJ = _AW // L


def _route1_body(idx_hbm, lp_hbm, hist_hbm, idx_v, lp_v, rc_v, stage_v):
    cid = lax.axis_index("c")
    sid = lax.axis_index("s")
    base = sid * _AW

    pltpu.sync_copy(idx_hbm.at[pl.ds(base, _AW)], idx_v)

    zero16 = jnp.zeros((L,), jnp.int32)
    lane = lax.iota(jnp.int32, L)

    def p1_step(j, rc):
        v = idx_v[pl.ds(j * L, L)]
        rc_v[...] = rc
        prev = plsc.load_gather(rc_v, [v])
        lp = zero16
        tot = zero16
        for e in range(E):
            m = v == e
            mi = jnp.where(m, 1, 0)
            cs = lax.cumsum(mi, axis=0)
            lp = jnp.where(m, cs - 1, lp)
            s = jnp.sum(mi)
            tot = jnp.where(lane == e, s, tot)
        lp_v[pl.ds(j * L, L)] = lp + prev
        return rc + tot

    hist = lax.fori_loop(0, _VJ, p1_step, zero16)
    stage_v[...] = hist

    @pl.when(cid == 0)
    def _():
        pltpu.sync_copy(lp_v, lp_hbm.at[pl.ds(base, _AW)])
        pltpu.sync_copy(stage_v, hist_hbm.at[sid])


def _route2_body(idx_hbm, sc_hbm, cap_hbm, lp_hbm, hist_hbm,
                 dstd_hbm, dstc_hbm, wts_hbm, cnt_hbm,
                 idx_v, sc_v, lp_v, allh_v, base_v,
                 outd_v, outc_v, outw_v, cap_v, stage_v):
    cid = lax.axis_index("c")
    sid = lax.axis_index("s")
    base = sid * _AW

    pltpu.sync_copy(idx_hbm.at[pl.ds(base, _AW)], idx_v)
    pltpu.sync_copy(sc_hbm.at[pl.ds(base, _AW)], sc_v)
    pltpu.sync_copy(cap_hbm, cap_v)
    pltpu.sync_copy(lp_hbm.at[pl.ds(base, _AW)], lp_v)
    pltpu.sync_copy(hist_hbm, allh_v)

    zero16 = jnp.zeros((L,), jnp.int32)
    bvec = zero16
    tvec = zero16
    for w in range(NS):
        row = allh_v[w]
        bvec = bvec + jnp.where(w < sid, row, zero16)
        tvec = tvec + row
    base_v[...] = bvec

    @pl.when(jnp.logical_and(cid == 0, sid == 0))
    def _():
        stage_v[...] = tvec
        pltpu.sync_copy(stage_v, cnt_hbm)

    capv = cap_v[...]

    def p2_step(j, carry):
        v = idx_v[pl.ds(j * L, L)]
        pos = lp_v[pl.ds(j * L, L)] + plsc.load_gather(base_v, [v])
        keep = pos < capv
        slot = jnp.where(keep, pos, CAP)
        outd_v[pl.ds(j * L, L)] = v * (CAP + 1) + slot
        outc_v[pl.ds(j * L, L)] = jnp.where(keep, v * CAP + pos, 0)
        sc = sc_v[pl.ds(j * L, L)]
        outw_v[pl.ds(j * L, L)] = jnp.where(keep, sc, jnp.zeros((L,), jnp.float32))
        return carry

    lax.fori_loop(0, _VJ, p2_step, 0)

    @pl.when(cid == 0)
    def _():
        pltpu.sync_copy(outd_v, dstd_hbm.at[pl.ds(base, _AW)])
        pltpu.sync_copy(outc_v, dstc_hbm.at[pl.ds(base, _AW)])
        pltpu.sync_copy(outw_v, wts_hbm.at[pl.ds(base, _AW)])


def _route(idx_flat, scores_flat, cap16):
    lp, hists = pl.kernel(
        _route1_body,
        out_type=(
            jax.ShapeDtypeStruct((A,), jnp.int32),
            jax.ShapeDtypeStruct((NS, L), jnp.int32),
        ),
        mesh=plsc.VectorSubcoreMesh(**_MESH),
        compiler_params=pltpu.CompilerParams(needs_layout_passes=False),
        scratch_types=[
            pltpu.VMEM((_AW,), jnp.int32),
            pltpu.VMEM((_AW,), jnp.int32),
            pltpu.VMEM((L,), jnp.int32),
            pltpu.VMEM((L,), jnp.int32),
        ],
    )(idx_flat)
    return pl.kernel(
        _route2_body,
        out_type=(
            jax.ShapeDtypeStruct((A,), jnp.int32),
            jax.ShapeDtypeStruct((A,), jnp.int32),
            jax.ShapeDtypeStruct((A,), jnp.float32),
            jax.ShapeDtypeStruct((L,), jnp.int32),
        ),
        mesh=plsc.VectorSubcoreMesh(**_MESH),
        compiler_params=pltpu.CompilerParams(needs_layout_passes=False),
        scratch_types=[
            pltpu.VMEM((_AW,), jnp.int32),
            pltpu.VMEM((_AW,), jnp.float32),
            pltpu.VMEM((_AW,), jnp.int32),
            pltpu.VMEM((NS, L), jnp.int32),
            pltpu.VMEM((L,), jnp.int32),
            pltpu.VMEM((_AW,), jnp.int32),
            pltpu.VMEM((_AW,), jnp.int32),
            pltpu.VMEM((_AW,), jnp.float32),
            pltpu.VMEM((L,), jnp.int32),
            pltpu.VMEM((L,), jnp.int32),
        ],
    )(idx_flat, scores_flat, cap16, lp, hists)


_TW = T // NW
_DR = 32
_NRD = _TW // _DR


def _dispatch_body(x_hbm, dstd_hbm, buf_hbm, dd_v, xv, ev, ov, sem):
    cid = lax.axis_index("c")
    sid = lax.axis_index("s")
    wid = sid * NC + cid
    abase = wid * _TW * K

    pltpu.sync_copy(dstd_hbm.at[pl.ds(abase, _TW * K)], dd_v)
    i0 = lax.iota(jnp.int32, L)

    def round_step(r, carry):
        off = r * _DR * K
        for h in range(_DR // L):
            sel = off + (h * L + i0) * K
            ev[pl.ds(h * L, L)] = plsc.load_gather(dd_v, [sel])
            ov[pl.ds(h * L, L)] = plsc.load_gather(dd_v, [sel + 1])
        pltpu.sync_copy(x_hbm.at[pl.ds(wid * _TW + r * _DR, _DR)], xv)
        pltpu.async_copy(xv, buf_hbm.at[ev], sem).wait()
        pltpu.async_copy(xv, buf_hbm.at[ov], sem).wait()
        return carry

    lax.fori_loop(0, _NRD, round_step, 0)


def _dispatch(x_flat, dstd):
    return pl.kernel(
        _dispatch_body,
        out_type=jax.ShapeDtypeStruct((E * (CAP + 1), D), jnp.float32),
        mesh=plsc.VectorSubcoreMesh(**_MESH),
        compiler_params=pltpu.CompilerParams(needs_layout_passes=False),
        scratch_types=[
            pltpu.VMEM((_TW * K,), jnp.int32),
            pltpu.VMEM((_DR, D), jnp.float32),
            pltpu.VMEM((_DR,), jnp.int32),
            pltpu.VMEM((_DR,), jnp.int32),
            pltpu.SemaphoreType.DMA,
        ],
    )(x_flat, dstd)


_BF = 512
_FB = F // _BF


def _ffn_body(buf_r, w1_r, b1_r, w2_r, b2_r, y_r):
    f = pl.program_id(1)
    x = buf_r[0]
    h = jax.nn.gelu(
        jnp.dot(x, w1_r[0], preferred_element_type=jnp.float32) + b1_r[0, 0])
    contrib = jnp.dot(h, w2_r[0], preferred_element_type=jnp.float32)

    @pl.when(f == 0)
    def _():
        y_r[0] = contrib

    @pl.when(f > 0)
    def _():
        y_r[0] += contrib

    @pl.when(f == _FB - 1)
    def _():
        y_r[0] += b2_r[0, 0]


def _ffn(buf3, W1, B1, W2, B2):
    return pl.pallas_call(
        _ffn_body,
        grid=(E, _FB),
        in_specs=[
            pl.BlockSpec((1, CAP, D), lambda e, f: (e, 0, 0)),
            pl.BlockSpec((1, D, _BF), lambda e, f: (e, 0, f)),
            pl.BlockSpec((1, 1, _BF), lambda e, f: (e, 0, f)),
            pl.BlockSpec((1, _BF, D), lambda e, f: (e, f, 0)),
            pl.BlockSpec((1, 1, D), lambda e, f: (e, 0, 0)),
        ],
        out_specs=pl.BlockSpec((1, CAP, D), lambda e, f: (e, 0, 0)),
        out_shape=jax.ShapeDtypeStruct((E, CAP, D), jnp.float32),
    )(buf3, W1, B1.reshape(E, 1, F), W2, B2.reshape(E, 1, D))


_CR = 16
_NCR = _TW // _CR
_CCH = D // L


def _combine_body(y_hbm, dstc_hbm, wts_hbm, out_hbm, dc_v, wt_v, idx_v,
                  rv, ov, sem):
    cid = lax.axis_index("c")
    sid = lax.axis_index("s")
    wid = sid * NC + cid
    abase = wid * _TW * K

    pltpu.sync_copy(dstc_hbm.at[pl.ds(abase, _TW * K)], dc_v)
    pltpu.sync_copy(wts_hbm.at[pl.ds(abase, _TW * K)], wt_v)
    zf = jnp.zeros((L,), jnp.float32)

    def round_step(r, carry):
        off = r * _CR * K
        for h in range(_CR * K // L):
            idx_v[pl.ds(h * L, L)] = dc_v[pl.ds(off + h * L, L)]
        pltpu.async_copy(y_hbm.at[idx_v], rv, sem).wait()
        for t in range(_CR):
            w0 = plsc.load_gather(wt_v, [jnp.full((L,), off + 2 * t, jnp.int32)])
            w1 = plsc.load_gather(wt_v, [jnp.full((L,), off + 2 * t + 1, jnp.int32)])
            k0 = w0 > 0.0
            k1 = w1 > 0.0

            def ch_step(c, carry2):
                r0 = rv[2 * t, pl.ds(c * L, L)]
                r1 = rv[2 * t + 1, pl.ds(c * L, L)]
                o = jnp.where(k0, w0 * r0, zf) + jnp.where(k1, w1 * r1, zf)
                ov[t, pl.ds(c * L, L)] = o
                return carry2

            lax.fori_loop(0, _CCH, ch_step, 0)
        pltpu.sync_copy(ov, out_hbm.at[pl.ds(wid * _TW + r * _CR, _CR)])
        return carry

    lax.fori_loop(0, _NCR, round_step, 0)


def _combine(y2, dstc, wts):
    return pl.kernel(
        _combine_body,
        out_type=jax.ShapeDtypeStruct((T, D), jnp.float32),
        mesh=plsc.VectorSubcoreMesh(**_MESH),
        compiler_params=pltpu.CompilerParams(needs_layout_passes=False),
        scratch_types=[
            pltpu.VMEM((_TW * K,), jnp.int32),
            pltpu.VMEM((_TW * K,), jnp.float32),
            pltpu.VMEM((_CR * K,), jnp.int32),
            pltpu.VMEM((_CR * K, D), jnp.float32),
            pltpu.VMEM((_CR, D), jnp.float32),
            pltpu.SemaphoreType.DMA,
        ],
    )(y2, dstc, wts)


def kernel(x_flat, idx_flat, scores_flat, capacity, W1, B1, W2, B2):
    cap16 = jnp.full((L,), capacity, jnp.int32)
    dstd, dstc, wts, counts16 = _route(idx_flat, scores_flat, cap16)
    buf = _dispatch(x_flat, dstd)
    y = _ffn(buf.reshape(E, CAP + 1, D), W1, B1, W2, B2)
    out = _combine(y.reshape(E * CAP, D), dstc, wts)
    return out, counts16[:E]
